# baseline (speedup 1.0000x reference)
import functools

import jax
import jax.numpy as jnp
from jax import lax
from jax.experimental import pallas as pl
from jax.experimental.pallas import tpu as pltpu
from jax.experimental.pallas import tpu_sc as plsc

B, L, EMBED = 16384, 50, 64
VROWS = 1002
TBL_WORDS = VROWS * EMBED
NC, NS = 2, 16
NW = NC * NS
SPT = B // NW
BLK = 4
SPG = 32
NBLK = SPG // BLK
NG = SPT // SPG

_mesh = plsc.VectorSubcoreMesh(core_axis_name="c", subcore_axis_name="s")


@functools.partial(
    pl.kernel,
    mesh=_mesh,
    out_type=jax.ShapeDtypeStruct((B, L, EMBED), jnp.float32),
    scratch_types=[
        pltpu.VMEM((TBL_WORDS,), jnp.float32),
        pltpu.VMEM((BLK, L, EMBED), jnp.float32),
        pltpu.VMEM((BLK, L, EMBED), jnp.float32),
        pltpu.VMEM((SPG * L + 16,), jnp.int32),
        pltpu.SemaphoreType.DMA,
        pltpu.SemaphoreType.DMA,
    ],
)
def _lookup(idx_hbm, tbl_hbm, out_hbm, tbl1, bufa, bufb, idx_v, sema, semb):
    wid = lax.axis_index("s") * NC + lax.axis_index("c")
    sb = wid * SPT

    pltpu.sync_copy(tbl_hbm, tbl1)

    def fill_block(buf, blk):
        def sample(s, carry):
            soff = (blk * BLK + s) * L
            ivs = [idx_v[pl.ds(soff + 16 * k, 16)] for k in range(4)]
            AHEAD = 8
            addrs = [ivs[i // 16][i % 16] for i in range(AHEAD)]
            for i in range(L):
                if i + AHEAD < L:
                    addrs.append(ivs[(i + AHEAD) // 16][(i + AHEAD) % 16])
                a = addrs[i]
                for k in range(4):
                    buf[s, i, pl.ds(k * 16, 16)] = tbl1[pl.ds(a + k * 16, 16)]
            return carry

        lax.fori_loop(0, BLK, sample, 0)

    def igroup(gi, carry):
        pltpu.sync_copy(
            idx_hbm.at[pl.ds((sb + gi * SPG) * L, SPG * L)],
            idx_v.at[pl.ds(0, SPG * L)],
        )
        for blk in range(NBLK):
            buf, sem = (bufa, sema) if blk % 2 == 0 else (bufb, semb)
            b = sb + gi * SPG + blk * BLK

            if blk < 2:
                @pl.when(gi > 0)
                def _():
                    pltpu.make_async_copy(
                        buf, out_hbm.at[pl.ds(b - 2 * BLK, BLK)], sem
                    ).wait()
            else:
                pltpu.make_async_copy(
                    buf, out_hbm.at[pl.ds(b - 2 * BLK, BLK)], sem
                ).wait()

            fill_block(buf, blk)
            pltpu.async_copy(buf, out_hbm.at[pl.ds(b, BLK)], sem)
        return carry

    lax.fori_loop(0, NG, igroup, 0)

    end = sb + SPT
    pltpu.make_async_copy(bufa, out_hbm.at[pl.ds(end - 2 * BLK, BLK)], sema).wait()
    pltpu.make_async_copy(bufb, out_hbm.at[pl.ds(end - BLK, BLK)], semb).wait()


def kernel(indices, table):
    idx64 = (indices.reshape(-1) * EMBED).astype(jnp.int32)
    return _lookup(idx64, table.reshape(-1))


# speedup vs baseline: 1.4414x; 1.4414x over previous
import functools

import jax
import jax.numpy as jnp
from jax import lax
from jax.experimental import pallas as pl
from jax.experimental.pallas import tpu as pltpu
from jax.experimental.pallas import tpu_sc as plsc

B, L, EMBED = 16384, 50, 64
VROWS = 1002
TBL_WORDS = VROWS * EMBED
NC, NS = 2, 16
NW = NC * NS
SPT = B // NW
BLK = 4
SPG = 32
NBLK = SPG // BLK
NG = SPT // SPG

_mesh = plsc.VectorSubcoreMesh(core_axis_name="c", subcore_axis_name="s")


@functools.partial(
    pl.kernel,
    mesh=_mesh,
    out_type=jax.ShapeDtypeStruct((B, L, EMBED), jnp.float32),
    scratch_types=[
        pltpu.VMEM((TBL_WORDS,), jnp.float32),
        pltpu.VMEM((BLK, L, EMBED), jnp.float32),
        pltpu.VMEM((BLK, L, EMBED), jnp.float32),
        pltpu.VMEM((SPG * L + 16,), jnp.int32),
        pltpu.SemaphoreType.DMA,
        pltpu.SemaphoreType.DMA,
    ],
)
def _lookup(idx_hbm, tbl_hbm, out_hbm, tbl1, bufa, bufb, idx_v, sema, semb):
    wid = lax.axis_index("s") * NC + lax.axis_index("c")
    sb = wid * SPT

    pltpu.sync_copy(tbl_hbm, tbl1)

    def fill_block(buf, blk):
        def sample(s, carry):
            soff = (blk * BLK + s) * L
            ivs = [idx_v[pl.ds(soff + 16 * k, 16)] for k in range(4)]
            AHEAD = 8
            addrs = [ivs[i // 16][i % 16] for i in range(AHEAD)]
            prev = None
            for i in range(L):
                if i + AHEAD < L:
                    addrs.append(ivs[(i + AHEAD) // 16][(i + AHEAD) % 16])
                a = addrs[i]
                cur = [tbl1[pl.ds(a + k * 16, 16)] for k in range(4)]
                if prev is not None:
                    for k in range(4):
                        buf[s, i - 1, pl.ds(k * 16, 16)] = prev[k]
                prev = cur
            for k in range(4):
                buf[s, L - 1, pl.ds(k * 16, 16)] = prev[k]
            return carry

        lax.fori_loop(0, BLK, sample, 0)

    def igroup(gi, carry):
        pltpu.sync_copy(
            idx_hbm.at[pl.ds((sb + gi * SPG) * L, SPG * L)],
            idx_v.at[pl.ds(0, SPG * L)],
        )
        for blk in range(NBLK):
            buf, sem = (bufa, sema) if blk % 2 == 0 else (bufb, semb)
            b = sb + gi * SPG + blk * BLK

            if blk < 2:
                @pl.when(gi > 0)
                def _():
                    pltpu.make_async_copy(
                        buf, out_hbm.at[pl.ds(b - 2 * BLK, BLK)], sem
                    ).wait()
            else:
                pltpu.make_async_copy(
                    buf, out_hbm.at[pl.ds(b - 2 * BLK, BLK)], sem
                ).wait()

            fill_block(buf, blk)
            pltpu.async_copy(buf, out_hbm.at[pl.ds(b, BLK)], sem)
        return carry

    lax.fori_loop(0, NG, igroup, 0)

    end = sb + SPT
    pltpu.make_async_copy(bufa, out_hbm.at[pl.ds(end - 2 * BLK, BLK)], sema).wait()
    pltpu.make_async_copy(bufb, out_hbm.at[pl.ds(end - BLK, BLK)], semb).wait()


def kernel(indices, table):
    idx64 = (indices.reshape(-1) * EMBED).astype(jnp.int32)
    return _lookup(idx64, table.reshape(-1))


# speedup vs baseline: 1.4483x; 1.0048x over previous
import functools

import jax
import jax.numpy as jnp
from jax import lax
from jax.experimental import pallas as pl
from jax.experimental.pallas import tpu as pltpu
from jax.experimental.pallas import tpu_sc as plsc

B, L, EMBED = 16384, 50, 64
VROWS = 1002
TBL_WORDS = VROWS * EMBED
NC, NS = 2, 16
NW = NC * NS
SPT = B // NW
BLK = 4
SPG = 32
NBLK = SPG // BLK
NG = SPT // SPG

_mesh = plsc.VectorSubcoreMesh(core_axis_name="c", subcore_axis_name="s")


@functools.partial(
    pl.kernel,
    mesh=_mesh,
    out_type=jax.ShapeDtypeStruct((B, L, EMBED), jnp.float32),
    scratch_types=[
        pltpu.VMEM((TBL_WORDS,), jnp.float32),
        pltpu.VMEM((BLK, L, EMBED), jnp.float32),
        pltpu.VMEM((BLK, L, EMBED), jnp.float32),
        pltpu.VMEM((SPG * L + 16,), jnp.int32),
        pltpu.SemaphoreType.DMA,
        pltpu.SemaphoreType.DMA,
    ],
)
def _lookup(idx_hbm, tbl_hbm, out_hbm, tbl1, bufa, bufb, idx_v, sema, semb):
    wid = lax.axis_index("s") * NC + lax.axis_index("c")
    sb = wid * SPT

    pltpu.sync_copy(tbl_hbm, tbl1)

    def fill_block(buf, blk):
        def sample(s, carry):
            soff = (blk * BLK + s) * L
            ivs = [idx_v[pl.ds(soff + 16 * k, 16)] for k in range(4)]
            AHEAD = 8
            DEPTH = 2
            addrs = [ivs[i // 16][i % 16] for i in range(AHEAD)]
            pending = []
            for i in range(L):
                if i + AHEAD < L:
                    addrs.append(ivs[(i + AHEAD) // 16][(i + AHEAD) % 16])
                a = addrs[i]
                pending.append((i, [tbl1[pl.ds(a + k * 16, 16)] for k in range(4)]))
                if len(pending) > DEPTH:
                    j, vals = pending.pop(0)
                    for k in range(4):
                        buf[s, j, pl.ds(k * 16, 16)] = vals[k]
            for j, vals in pending:
                for k in range(4):
                    buf[s, j, pl.ds(k * 16, 16)] = vals[k]
            return carry

        lax.fori_loop(0, BLK, sample, 0)

    def igroup(gi, carry):
        pltpu.sync_copy(
            idx_hbm.at[pl.ds((sb + gi * SPG) * L, SPG * L)],
            idx_v.at[pl.ds(0, SPG * L)],
        )
        for blk in range(NBLK):
            buf, sem = (bufa, sema) if blk % 2 == 0 else (bufb, semb)
            b = sb + gi * SPG + blk * BLK

            if blk < 2:
                @pl.when(gi > 0)
                def _():
                    pltpu.make_async_copy(
                        buf, out_hbm.at[pl.ds(b - 2 * BLK, BLK)], sem
                    ).wait()
            else:
                pltpu.make_async_copy(
                    buf, out_hbm.at[pl.ds(b - 2 * BLK, BLK)], sem
                ).wait()

            fill_block(buf, blk)
            pltpu.async_copy(buf, out_hbm.at[pl.ds(b, BLK)], sem)
        return carry

    lax.fori_loop(0, NG, igroup, 0)

    end = sb + SPT
    pltpu.make_async_copy(bufa, out_hbm.at[pl.ds(end - 2 * BLK, BLK)], sema).wait()
    pltpu.make_async_copy(bufb, out_hbm.at[pl.ds(end - BLK, BLK)], semb).wait()


def kernel(indices, table):
    idx64 = (indices.reshape(-1) * EMBED).astype(jnp.int32)
    return _lookup(idx64, table.reshape(-1))
